# baseline (speedup 1.0000x reference)
import functools
import math

import jax
import jax.numpy as jnp
from jax import lax
from jax.experimental import pallas as pl
from jax.experimental.pallas import tpu as pltpu
from jax.experimental.pallas import tpu_sc as plsc

_CUTOFF = 10.0
_LOG2 = math.log(2.0)

_NC = 2
_NS = 16
_L = 16
_NW = _NC * _NS


def _ssp(v):
    return jnp.maximum(v, 0.0) + jnp.log(1.0 + jnp.exp(-jnp.abs(v))) - _LOG2


def _ssp_fast(v):
    return jnp.log(0.5 + 0.5 * jnp.exp(v))


def _filter_body(eat_ref, ew_ref, w1t_ref, b1_ref, w2t_ref, b2_ref, wm_ref):
    a = lax.dot_general(eat_ref[...], w1t_ref[...], (((0,), (0,)), ((), ())),
                        preferred_element_type=jnp.float32)
    a = _ssp_fast(a + b1_ref[...])
    w = jnp.dot(a, w2t_ref[...],
                preferred_element_type=jnp.float32) + b2_ref[...]
    cr = 0.5 * (jnp.cos(ew_ref[...] * (math.pi / _CUTOFF)) + 1.0)
    wm_ref[...] = (w * jnp.swapaxes(cr, 0, 1)).astype(jnp.bfloat16)


def _filter_call(ea_t, ew_row, w1t, b1, w2t, b2, block_e):
    g, e = ea_t.shape
    fl = w1t.shape[1]
    nb = e // block_e
    return pl.pallas_call(
        _filter_body,
        grid=(nb,),
        in_specs=[
            pl.BlockSpec((g, block_e), lambda i: (0, i)),
            pl.BlockSpec((1, block_e), lambda i: (0, i)),
            pl.BlockSpec((g, fl), lambda i: (0, 0)),
            pl.BlockSpec((1, fl), lambda i: (0, 0)),
            pl.BlockSpec((fl, fl), lambda i: (0, 0)),
            pl.BlockSpec((1, fl), lambda i: (0, 0)),
        ],
        out_specs=pl.BlockSpec((block_e, fl), lambda i: (i, 0)),
        out_shape=jax.ShapeDtypeStruct((e, fl), jnp.bfloat16),
    )(ea_t, ew_row, w1t, b1, w2t, b2)


def _lin1_body(x_ref, wt_ref, out_ref):
    out_ref[...] = jnp.dot(x_ref[...], wt_ref[...],
                           preferred_element_type=jnp.float32
                           ).astype(jnp.bfloat16)


def _lin1_call(x, lin1t, block_n):
    n, h = x.shape
    fl = lin1t.shape[1]
    grid = n // block_n
    return pl.pallas_call(
        _lin1_body,
        grid=(grid,),
        in_specs=[
            pl.BlockSpec((block_n, h), lambda i: (i, 0)),
            pl.BlockSpec((h, fl), lambda i: (0, 0)),
        ],
        out_specs=pl.BlockSpec((block_n, fl), lambda i: (i, 0)),
        out_shape=jax.ShapeDtypeStruct((n, fl), jnp.bfloat16),
    )(x, lin1t)


def _sc_aggregate(h32, wm32, ei, n_pad, fl, ew, ch, nch):
    rps = n_pad // _NS
    flw = fl // 2

    mesh = plsc.VectorSubcoreMesh(core_axis_name="c", subcore_axis_name="s")

    @functools.partial(
        pl.kernel,
        out_type=jax.ShapeDtypeStruct((_NC, n_pad, fl), jnp.float32),
        mesh=mesh,
        compiler_params=pltpu.CompilerParams(use_tc_tiling_on_sc=False,
                                             needs_layout_passes=False),
        scratch_types=[
            pltpu.VMEM((nch, ch), jnp.int32),
            pltpu.VMEM((nch, ch), jnp.int32),
            pltpu.VMEM((ch, flw), jnp.int32),
            pltpu.VMEM((ch, flw), jnp.int32),
            pltpu.VMEM((ch, flw), jnp.int32),
            pltpu.VMEM((ch, flw), jnp.int32),
            pltpu.VMEM((ch, fl), jnp.float32),
            pltpu.VMEM((ch, fl), jnp.float32),
            pltpu.VMEM_SHARED((n_pad, fl), jnp.float32),
            pltpu.SemaphoreType.DMA,
            pltpu.SemaphoreType.DMA,
            pltpu.SemaphoreType.DMA,
            pltpu.SemaphoreType.DMA,
            pltpu.SemaphoreType.DMA,
            pltpu.SemaphoreType.DMA,
        ],
    )
    def sc_kernel(h_hbm, wm_hbm, ei_hbm, out_hbm,
                  src_v, dst_v, rows0, rows1, wmb0, wmb1, msg0, msg1, agg_sh,
                  semw0, semw1, semg0, semg1, sema0, sema1):
        c = lax.axis_index("c")
        s = lax.axis_index("s")
        wid = s * _NC + c

        def zero_row(r, _):
            for j in range(fl // _L):
                msg0[r, pl.ds(j * _L, _L)] = jnp.zeros((_L,), jnp.float32)
            return 0
        lax.fori_loop(0, ch, zero_row, 0)
        for k in range(rps // ch):
            pltpu.sync_copy(msg0, agg_sh.at[pl.ds(s * rps + k * ch, ch)])

        pltpu.sync_copy(ei_hbm.at[0, wid], src_v)
        pltpu.sync_copy(ei_hbm.at[1, wid], dst_v)
        plsc.subcore_barrier()

        def issue(i, wmb, rowsb, semw, semg):
            base = wid * ew + i * ch
            pltpu.async_copy(wm_hbm.at[pl.ds(base, ch)], wmb, semw)
            pltpu.async_copy(h_hbm.at[src_v.at[i]], rowsb, semg)

        def wait_in(wmb, rowsb, semw, semg):
            pltpu.make_async_copy(wm_hbm.at[pl.ds(0, ch)], wmb, semw).wait()
            pltpu.make_async_copy(wm_hbm.at[pl.ds(0, ch)], rowsb, semg).wait()

        def wait_add(msgb, sema):
            pltpu.make_async_copy(out_hbm.at[0, pl.ds(0, ch)], msgb,
                                  sema).wait()

        mask = jnp.int32(-65536)

        def mul_rows(msgb, wmb, rowsb):
            def row_body(r, _):
                for g in range(flw // _L):
                    sl = pl.ds(g * _L, _L)
                    ww = wmb[r, sl]
                    hw = rowsb[r, sl]
                    wlo = plsc.bitcast(lax.shift_left(ww, 16), jnp.float32)
                    hlo = plsc.bitcast(lax.shift_left(hw, 16), jnp.float32)
                    whi = plsc.bitcast(ww & mask, jnp.float32)
                    hhi = plsc.bitcast(hw & mask, jnp.float32)
                    msgb[r, pl.ds(2 * g * _L, _L)] = wlo * hlo
                    msgb[r, pl.ds((2 * g + 1) * _L, _L)] = whi * hhi
                return 0
            lax.fori_loop(0, ch, row_body, 0)

        issue(0, wmb0, rows0, semw0, semg0)

        def body2(gidx, _):
            i0 = gidx * 2

            @pl.when(i0 > 0)
            def _():
                wait_add(msg0, sema0)
            issue(i0 + 1, wmb1, rows1, semw1, semg1)
            wait_in(wmb0, rows0, semw0, semg0)
            mul_rows(msg0, wmb0, rows0)
            pltpu.async_copy(msg0, agg_sh.at[dst_v.at[i0]], sema0, add=True)

            @pl.when(i0 > 0)
            def _():
                wait_add(msg1, sema1)

            @pl.when(i0 + 2 < nch)
            def _():
                issue(i0 + 2, wmb0, rows0, semw0, semg0)
            wait_in(wmb1, rows1, semw1, semg1)
            mul_rows(msg1, wmb1, rows1)
            pltpu.async_copy(msg1, agg_sh.at[dst_v.at[i0 + 1]], sema1,
                             add=True)
            return 0
        lax.fori_loop(0, nch // 2, body2, 0)

        wait_add(msg0, sema0)
        wait_add(msg1, sema1)

        plsc.subcore_barrier()
        pltpu.sync_copy(agg_sh.at[pl.ds(s * rps, rps)],
                        out_hbm.at[c, pl.ds(s * rps, rps)])

    return sc_kernel(h32, wm32, ei)


def _tail_body(p_ref, lin2t_ref, b2_ref, lint_ref, b_ref, out_ref):
    agg = p_ref[0] + p_ref[1]
    v = jnp.dot(agg, lin2t_ref[...], preferred_element_type=jnp.float32)
    v = _ssp(v + b2_ref[...])
    out_ref[...] = jnp.dot(v, lint_ref[...],
                           preferred_element_type=jnp.float32) + b_ref[...]


def _tail_call(partial, lin2t, lin2_b, lint, lin_b, n, block_n):
    fl = partial.shape[2]
    h = lint.shape[1]
    grid = n // block_n
    return pl.pallas_call(
        _tail_body,
        grid=(grid,),
        in_specs=[
            pl.BlockSpec((_NC, block_n, fl), lambda i: (0, i, 0)),
            pl.BlockSpec((fl, h), lambda i: (0, 0)),
            pl.BlockSpec((1, h), lambda i: (0, 0)),
            pl.BlockSpec((h, h), lambda i: (0, 0)),
            pl.BlockSpec((1, h), lambda i: (0, 0)),
        ],
        out_specs=pl.BlockSpec((block_n, h), lambda i: (i, 0)),
        out_shape=jax.ShapeDtypeStruct((n, h), jnp.float32),
    )(partial, lin2t, lin2_b, lint, lin_b)


def kernel(x, edge_index, edge_weight, edge_attr,
           mlp_w1, mlp_b1, mlp_w2, mlp_b2,
           lin1_w, lin2_w, lin2_b, lin_w, lin_b):
    n, h = x.shape
    e = edge_index.shape[1]
    fl = mlp_w1.shape[0]

    ew = e // _NW
    ch = 40
    nch = ew // ch
    be = 2560

    wm = _filter_call(edge_attr.T, edge_weight.reshape(1, e),
                      mlp_w1.T, mlp_b1.reshape(1, fl),
                      mlp_w2.T, mlp_b2.reshape(1, fl), block_e=be)
    hmat = _lin1_call(x, lin1_w.T, block_n=1000)

    wm32 = lax.bitcast_convert_type(wm.reshape(e, fl // 2, 2), jnp.int32)
    h32 = lax.bitcast_convert_type(hmat.reshape(n, fl // 2, 2), jnp.int32)

    n_pad = 10240
    ei = edge_index.reshape(2, _NW, nch, ch)
    partial = _sc_aggregate(h32, wm32, ei, n_pad, fl, ew, ch, nch)

    rho = [32 * (p // 32) + (2 * (p % 32) if p % 32 < 16
                             else 2 * (p % 32 - 16) + 1)
           for p in range(fl)]
    lin2t_perm = lin2_w.T[jnp.array(rho, jnp.int32), :]

    return _tail_call(partial, lin2t_perm, lin2_b.reshape(1, h),
                      lin_w.T, lin_b.reshape(1, h), n, block_n=1000)


# speedup vs baseline: 1.9165x; 1.9165x over previous
import functools
import math

import jax
import jax.numpy as jnp
from jax import lax
from jax.experimental import pallas as pl
from jax.experimental.pallas import tpu as pltpu
from jax.experimental.pallas import tpu_sc as plsc

_CUTOFF = 10.0
_LOG2 = math.log(2.0)

_NC = 2
_NS = 16
_L = 16
_NW = _NC * _NS


def _ssp(v):
    return jnp.maximum(v, 0.0) + jnp.log(1.0 + jnp.exp(-jnp.abs(v))) - _LOG2


def _ssp_fast(v):
    return jnp.log(0.5 + 0.5 * jnp.exp(v))


def _filter_body(eat_ref, ew_ref, w1t_ref, b1_ref, w2t_ref, b2_ref, wm_ref):
    a = lax.dot_general(eat_ref[...], w1t_ref[...], (((0,), (0,)), ((), ())),
                        preferred_element_type=jnp.float32)
    a = _ssp_fast(a + b1_ref[...])
    w = jnp.dot(a, w2t_ref[...],
                preferred_element_type=jnp.float32) + b2_ref[...]
    cr = 0.5 * (jnp.cos(ew_ref[...] * (math.pi / _CUTOFF)) + 1.0)
    wm_ref[...] = (w * jnp.swapaxes(cr, 0, 1)).astype(jnp.bfloat16)


def _filter_call(ea_t, ew_row, w1t, b1, w2t, b2, block_e):
    g, e = ea_t.shape
    fl = w1t.shape[1]
    nb = e // block_e
    return pl.pallas_call(
        _filter_body,
        grid=(nb,),
        in_specs=[
            pl.BlockSpec((g, block_e), lambda i: (0, i)),
            pl.BlockSpec((1, block_e), lambda i: (0, i)),
            pl.BlockSpec((g, fl), lambda i: (0, 0)),
            pl.BlockSpec((1, fl), lambda i: (0, 0)),
            pl.BlockSpec((fl, fl), lambda i: (0, 0)),
            pl.BlockSpec((1, fl), lambda i: (0, 0)),
        ],
        out_specs=pl.BlockSpec((block_e, fl), lambda i: (i, 0)),
        out_shape=jax.ShapeDtypeStruct((e, fl), jnp.bfloat16),
    )(ea_t, ew_row, w1t, b1, w2t, b2)


def _lin1_body(x_ref, wt_ref, out_ref):
    out_ref[...] = jnp.dot(x_ref[...], wt_ref[...],
                           preferred_element_type=jnp.float32
                           ).astype(jnp.bfloat16)


def _lin1_call(x, lin1t, block_n):
    n, h = x.shape
    fl = lin1t.shape[1]
    grid = n // block_n
    return pl.pallas_call(
        _lin1_body,
        grid=(grid,),
        in_specs=[
            pl.BlockSpec((block_n, h), lambda i: (i, 0)),
            pl.BlockSpec((h, fl), lambda i: (0, 0)),
        ],
        out_specs=pl.BlockSpec((block_n, fl), lambda i: (i, 0)),
        out_shape=jax.ShapeDtypeStruct((n, fl), jnp.bfloat16),
    )(x, lin1t)


def _sc_aggregate(h, wm, ei, n_pad, fl, ew, ch, nch):
    rps = n_pad // _NS
    flw = fl // 2

    mesh = plsc.VectorSubcoreMesh(core_axis_name="c", subcore_axis_name="s")

    @functools.partial(
        pl.kernel,
        out_type=jax.ShapeDtypeStruct((_NC, n_pad, fl), jnp.float32),
        mesh=mesh,
        compiler_params=pltpu.CompilerParams(use_tc_tiling_on_sc=False,
                                             needs_layout_passes=False),
        scratch_types=[
            pltpu.VMEM((nch, ch), jnp.int32),
            pltpu.VMEM((nch, ch), jnp.int32),
            pltpu.VMEM((ch, fl), jnp.bfloat16),
            pltpu.VMEM((ch, fl), jnp.bfloat16),
            pltpu.VMEM((ch, fl), jnp.bfloat16),
            pltpu.VMEM((ch, fl), jnp.bfloat16),
            pltpu.VMEM((ch, fl), jnp.float32),
            pltpu.VMEM((ch, fl), jnp.float32),
            pltpu.VMEM_SHARED((n_pad, fl), jnp.float32),
            pltpu.SemaphoreType.DMA,
            pltpu.SemaphoreType.DMA,
            pltpu.SemaphoreType.DMA,
            pltpu.SemaphoreType.DMA,
            pltpu.SemaphoreType.DMA,
            pltpu.SemaphoreType.DMA,
        ],
    )
    def sc_kernel(h_hbm, wm_hbm, ei_hbm, out_hbm,
                  src_v, dst_v, rows0, rows1, wmb0, wmb1, msg0, msg1, agg_sh,
                  semw0, semw1, semg0, semg1, sema0, sema1):
        c = lax.axis_index("c")
        s = lax.axis_index("s")
        wid = s * _NC + c

        def zero_row(r, _):
            for j in range(fl // _L):
                msg0[r, pl.ds(j * _L, _L)] = jnp.zeros((_L,), jnp.float32)
            return 0
        lax.fori_loop(0, ch, zero_row, 0)
        for k in range(rps // ch):
            pltpu.sync_copy(msg0, agg_sh.at[pl.ds(s * rps + k * ch, ch)])

        pltpu.sync_copy(ei_hbm.at[0, wid], src_v)
        pltpu.sync_copy(ei_hbm.at[1, wid], dst_v)
        plsc.subcore_barrier()

        def issue(i, wmb, rowsb, semw, semg):
            base = wid * ew + i * ch
            pltpu.async_copy(wm_hbm.at[pl.ds(base, ch)], wmb, semw)
            pltpu.async_copy(h_hbm.at[src_v.at[i]], rowsb, semg)

        def wait_in(wmb, rowsb, semw, semg):
            pltpu.make_async_copy(wm_hbm.at[pl.ds(0, ch)], wmb, semw).wait()
            pltpu.make_async_copy(wm_hbm.at[pl.ds(0, ch)], rowsb, semg).wait()

        def wait_add(msgb, sema):
            pltpu.make_async_copy(out_hbm.at[0, pl.ds(0, ch)], msgb,
                                  sema).wait()

        mask = jnp.int32(-65536)

        def mul_rows(msgb, wmb, rowsb):
            def row_body(r, _):
                for g in range(flw // _L):
                    sl = pl.ds(g * 2 * _L, 2 * _L)
                    ww = plsc.bitcast(wmb[r, sl], jnp.int32)
                    hw = plsc.bitcast(rowsb[r, sl], jnp.int32)
                    wlo = plsc.bitcast(lax.shift_left(ww, 16), jnp.float32)
                    hlo = plsc.bitcast(lax.shift_left(hw, 16), jnp.float32)
                    whi = plsc.bitcast(ww & mask, jnp.float32)
                    hhi = plsc.bitcast(hw & mask, jnp.float32)
                    msgb[r, pl.ds(2 * g * _L, _L)] = wlo * hlo
                    msgb[r, pl.ds((2 * g + 1) * _L, _L)] = whi * hhi
                return 0
            lax.fori_loop(0, ch, row_body, 0)

        issue(0, wmb0, rows0, semw0, semg0)

        def body2(gidx, _):
            i0 = gidx * 2

            @pl.when(i0 > 0)
            def _():
                wait_add(msg0, sema0)
            issue(i0 + 1, wmb1, rows1, semw1, semg1)
            wait_in(wmb0, rows0, semw0, semg0)
            mul_rows(msg0, wmb0, rows0)
            pltpu.async_copy(msg0, agg_sh.at[dst_v.at[i0]], sema0, add=True)

            @pl.when(i0 > 0)
            def _():
                wait_add(msg1, sema1)

            @pl.when(i0 + 2 < nch)
            def _():
                issue(i0 + 2, wmb0, rows0, semw0, semg0)
            wait_in(wmb1, rows1, semw1, semg1)
            mul_rows(msg1, wmb1, rows1)
            pltpu.async_copy(msg1, agg_sh.at[dst_v.at[i0 + 1]], sema1,
                             add=True)
            return 0
        lax.fori_loop(0, nch // 2, body2, 0)

        wait_add(msg0, sema0)
        wait_add(msg1, sema1)

        plsc.subcore_barrier()
        pltpu.sync_copy(agg_sh.at[pl.ds(s * rps, rps)],
                        out_hbm.at[c, pl.ds(s * rps, rps)])

    return sc_kernel(h, wm, ei)


def _tail_body(p_ref, lin2t_ref, b2_ref, lint_ref, b_ref, out_ref):
    agg = p_ref[0] + p_ref[1]
    v = jnp.dot(agg, lin2t_ref[...], preferred_element_type=jnp.float32)
    v = _ssp(v + b2_ref[...])
    out_ref[...] = jnp.dot(v, lint_ref[...],
                           preferred_element_type=jnp.float32) + b_ref[...]


def _tail_call(partial, lin2t, lin2_b, lint, lin_b, n, block_n):
    fl = partial.shape[2]
    h = lint.shape[1]
    grid = n // block_n
    return pl.pallas_call(
        _tail_body,
        grid=(grid,),
        in_specs=[
            pl.BlockSpec((_NC, block_n, fl), lambda i: (0, i, 0)),
            pl.BlockSpec((fl, h), lambda i: (0, 0)),
            pl.BlockSpec((1, h), lambda i: (0, 0)),
            pl.BlockSpec((h, h), lambda i: (0, 0)),
            pl.BlockSpec((1, h), lambda i: (0, 0)),
        ],
        out_specs=pl.BlockSpec((block_n, h), lambda i: (i, 0)),
        out_shape=jax.ShapeDtypeStruct((n, h), jnp.float32),
    )(partial, lin2t, lin2_b, lint, lin_b)


def kernel(x, edge_index, edge_weight, edge_attr,
           mlp_w1, mlp_b1, mlp_w2, mlp_b2,
           lin1_w, lin2_w, lin2_b, lin_w, lin_b):
    n, h = x.shape
    e = edge_index.shape[1]
    fl = mlp_w1.shape[0]

    ew = e // _NW
    ch = 40
    nch = ew // ch
    be = 2560

    wm = _filter_call(edge_attr.T, edge_weight.reshape(1, e),
                      mlp_w1.T, mlp_b1.reshape(1, fl),
                      mlp_w2.T, mlp_b2.reshape(1, fl), block_e=be)
    hmat = _lin1_call(x, lin1_w.T, block_n=1000)

    n_pad = 10240
    ei = edge_index.reshape(2, _NW, nch, ch)
    partial = _sc_aggregate(hmat, wm, ei, n_pad, fl, ew, ch, nch)

    rho = [32 * (p // 32) + (2 * (p % 32) if p % 32 < 16
                             else 2 * (p % 32 - 16) + 1)
           for p in range(fl)]
    lin2t_perm = lin2_w.T[jnp.array(rho, jnp.int32), :]

    return _tail_call(partial, lin2t_perm, lin2_b.reshape(1, h),
                      lin_w.T, lin_b.reshape(1, h), n, block_n=1000)


# speedup vs baseline: 3.6142x; 1.8858x over previous
import functools
import math

import jax
import jax.numpy as jnp
from jax import lax
from jax.experimental import pallas as pl
from jax.experimental.pallas import tpu as pltpu
from jax.experimental.pallas import tpu_sc as plsc

_CUTOFF = 10.0
_LOG2 = math.log(2.0)

_NC = 2
_NS = 16
_L = 16
_NW = _NC * _NS


def _ssp(v):
    return jnp.maximum(v, 0.0) + jnp.log(1.0 + jnp.exp(-jnp.abs(v))) - _LOG2


def _ssp_fast(v):
    return jnp.log(0.5 + 0.5 * jnp.exp(v))


def _filter_body(eat_ref, ew_ref, w1t_ref, b1_ref, w2t_ref, b2_ref,
                 wm_ref, c_ref):
    a = lax.dot_general(eat_ref[...], w1t_ref[...], (((0,), (0,)), ((), ())),
                        preferred_element_type=jnp.float32)
    a = _ssp_fast(a + b1_ref[...])
    wm_ref[...] = jnp.dot(a, w2t_ref[...],
                          preferred_element_type=jnp.float32) + b2_ref[...]
    c_ref[...] = 0.5 * (jnp.cos(ew_ref[...] * (math.pi / _CUTOFF)) + 1.0)


def _filter_call(ea_t, ew3, w1t, b1, w2t, b2, block_e):
    g, e = ea_t.shape
    fl = w1t.shape[1]
    nb = e // block_e
    sub = block_e // 8
    return pl.pallas_call(
        _filter_body,
        grid=(nb,),
        in_specs=[
            pl.BlockSpec((g, block_e), lambda i: (0, i)),
            pl.BlockSpec((1, 8, sub), lambda i: (i, 0, 0)),
            pl.BlockSpec((g, fl), lambda i: (0, 0)),
            pl.BlockSpec((1, fl), lambda i: (0, 0)),
            pl.BlockSpec((fl, fl), lambda i: (0, 0)),
            pl.BlockSpec((1, fl), lambda i: (0, 0)),
        ],
        out_specs=[
            pl.BlockSpec((block_e, fl), lambda i: (i, 0)),
            pl.BlockSpec((1, 8, sub), lambda i: (i, 0, 0)),
        ],
        out_shape=[
            jax.ShapeDtypeStruct((e, fl), jnp.float32),
            jax.ShapeDtypeStruct((nb, 8, sub), jnp.float32),
        ],
    )(ea_t, ew3, w1t, b1, w2t, b2)


def _lin1_body(x_ref, wt_ref, out_ref):
    out_ref[...] = jnp.dot(x_ref[...], wt_ref[...],
                           preferred_element_type=jnp.float32)


def _lin1_call(x, lin1t, block_n):
    n, h = x.shape
    fl = lin1t.shape[1]
    grid = n // block_n
    return pl.pallas_call(
        _lin1_body,
        grid=(grid,),
        in_specs=[
            pl.BlockSpec((block_n, h), lambda i: (i, 0)),
            pl.BlockSpec((h, fl), lambda i: (0, 0)),
        ],
        out_specs=pl.BlockSpec((block_n, fl), lambda i: (i, 0)),
        out_shape=jax.ShapeDtypeStruct((n, fl), jnp.float32),
    )(x, lin1t)


def _sc_aggregate(h, wm, ei, ced, n_pad, fl, ew, ch, nch):
    rps = n_pad // _NS

    mesh = plsc.VectorSubcoreMesh(core_axis_name="c", subcore_axis_name="s")

    @functools.partial(
        pl.kernel,
        out_type=jax.ShapeDtypeStruct((_NC, n_pad, fl), jnp.float32),
        mesh=mesh,
        compiler_params=pltpu.CompilerParams(use_tc_tiling_on_sc=False,
                                             needs_layout_passes=False),
        scratch_types=[
            pltpu.VMEM((nch, ch), jnp.int32),
            pltpu.VMEM((nch, ch), jnp.int32),
            pltpu.VMEM((ch, fl), jnp.float32),
            pltpu.VMEM((ch, fl), jnp.float32),
            pltpu.VMEM((ch, fl), jnp.float32),
            pltpu.VMEM((ch, fl), jnp.float32),
            pltpu.VMEM((ch + 8,), jnp.float32),
            pltpu.VMEM((ch + 8,), jnp.float32),
            pltpu.VMEM_SHARED((n_pad, fl), jnp.float32),
            pltpu.SemaphoreType.DMA,
            pltpu.SemaphoreType.DMA,
            pltpu.SemaphoreType.DMA,
            pltpu.SemaphoreType.DMA,
            pltpu.SemaphoreType.DMA,
            pltpu.SemaphoreType.DMA,
            pltpu.SemaphoreType.DMA,
            pltpu.SemaphoreType.DMA,
        ],
    )
    def sc_kernel(h_hbm, wm_hbm, ei_hbm, c_hbm, out_hbm,
                  src_v, dst_v, rows0, rows1, msg0, msg1, cb0, cb1, agg_sh,
                  semw0, semw1, semg0, semg1, semc0, semc1, sema0, sema1):
        c = lax.axis_index("c")
        s = lax.axis_index("s")
        wid = s * _NC + c

        def zero_row(r, _):
            for j in range(fl // _L):
                msg0[r, pl.ds(j * _L, _L)] = jnp.zeros((_L,), jnp.float32)
            return 0
        lax.fori_loop(0, ch, zero_row, 0)
        for k in range(rps // ch):
            pltpu.sync_copy(msg0, agg_sh.at[pl.ds(s * rps + k * ch, ch)])

        pltpu.sync_copy(ei_hbm.at[0, wid], src_v)
        pltpu.sync_copy(ei_hbm.at[1, wid], dst_v)
        plsc.subcore_barrier()

        def issue(i, msgb, rowsb, cb, semw, semg, semc):
            base = wid * ew + i * ch
            pltpu.async_copy(wm_hbm.at[pl.ds(base, ch)], msgb, semw)
            pltpu.async_copy(h_hbm.at[src_v.at[i]], rowsb, semg)
            pltpu.async_copy(c_hbm.at[wid, i], cb.at[pl.ds(0, ch)], semc)

        def wait_in(msgb, rowsb, cb, semw, semg, semc):
            pltpu.make_async_copy(wm_hbm.at[pl.ds(0, ch)], msgb, semw).wait()
            pltpu.make_async_copy(wm_hbm.at[pl.ds(0, ch)], rowsb, semg).wait()
            pltpu.make_async_copy(c_hbm.at[0, 0], cb.at[pl.ds(0, ch)],
                                  semc).wait()

        def wait_add(msgb, sema):
            pltpu.make_async_copy(wm_hbm.at[pl.ds(0, ch)], msgb, sema).wait()

        def mul_rows(msgb, rowsb, cb):
            def grp_body(g, _):
                cvec = cb[pl.ds(g * 8, _L)]
                for k in range(8):
                    cs = cvec[k]
                    r = g * 8 + k
                    for j in range(fl // _L):
                        sl = pl.ds(j * _L, _L)
                        msgb[r, sl] = rowsb[r, sl] * msgb[r, sl] * cs
                return 0
            lax.fori_loop(0, ch // 8, grp_body, 0)

        issue(0, msg0, rows0, cb0, semw0, semg0, semc0)

        def body2(gidx, _):
            i0 = gidx * 2

            @pl.when(i0 > 0)
            def _():
                wait_add(msg0, sema0)
            issue(i0 + 1, msg1, rows1, cb1, semw1, semg1, semc1)
            wait_in(msg0, rows0, cb0, semw0, semg0, semc0)
            mul_rows(msg0, rows0, cb0)
            pltpu.async_copy(msg0, agg_sh.at[dst_v.at[i0]], sema0, add=True)

            @pl.when(i0 > 0)
            def _():
                wait_add(msg1, sema1)

            @pl.when(i0 + 2 < nch)
            def _():
                issue(i0 + 2, msg0, rows0, cb0, semw0, semg0, semc0)
            wait_in(msg1, rows1, cb1, semw1, semg1, semc1)
            mul_rows(msg1, rows1, cb1)
            pltpu.async_copy(msg1, agg_sh.at[dst_v.at[i0 + 1]], sema1,
                             add=True)
            return 0
        lax.fori_loop(0, nch // 2, body2, 0)

        wait_add(msg0, sema0)
        wait_add(msg1, sema1)

        plsc.subcore_barrier()
        pltpu.sync_copy(agg_sh.at[pl.ds(s * rps, rps)],
                        out_hbm.at[c, pl.ds(s * rps, rps)])

    return sc_kernel(h, wm, ei, ced)


def _tail_body(p_ref, lin2t_ref, b2_ref, lint_ref, b_ref, out_ref):
    agg = p_ref[0] + p_ref[1]
    v = jnp.dot(agg, lin2t_ref[...], preferred_element_type=jnp.float32)
    v = _ssp(v + b2_ref[...])
    out_ref[...] = jnp.dot(v, lint_ref[...],
                           preferred_element_type=jnp.float32) + b_ref[...]


def _tail_call(partial, lin2t, lin2_b, lint, lin_b, n, block_n):
    fl = partial.shape[2]
    h = lint.shape[1]
    grid = n // block_n
    return pl.pallas_call(
        _tail_body,
        grid=(grid,),
        in_specs=[
            pl.BlockSpec((_NC, block_n, fl), lambda i: (0, i, 0)),
            pl.BlockSpec((fl, h), lambda i: (0, 0)),
            pl.BlockSpec((1, h), lambda i: (0, 0)),
            pl.BlockSpec((h, h), lambda i: (0, 0)),
            pl.BlockSpec((1, h), lambda i: (0, 0)),
        ],
        out_specs=pl.BlockSpec((block_n, h), lambda i: (i, 0)),
        out_shape=jax.ShapeDtypeStruct((n, h), jnp.float32),
    )(partial, lin2t, lin2_b, lint, lin_b)


def kernel(x, edge_index, edge_weight, edge_attr,
           mlp_w1, mlp_b1, mlp_w2, mlp_b2,
           lin1_w, lin2_w, lin2_b, lin_w, lin_b):
    n, h = x.shape
    e = edge_index.shape[1]
    fl = mlp_w1.shape[0]

    ew = e // _NW
    ch = 40
    nch = ew // ch
    be = 2560

    wm, c3 = _filter_call(edge_attr.T, edge_weight.reshape(e // be, 8, be // 8),
                          mlp_w1.T, mlp_b1.reshape(1, fl),
                          mlp_w2.T, mlp_b2.reshape(1, fl), block_e=be)
    hmat = _lin1_call(x, lin1_w.T, block_n=1000)

    n_pad = 10240
    ei = edge_index.reshape(2, _NW, nch, ch)
    ced = c3.reshape(_NW, nch, ch)
    partial = _sc_aggregate(hmat, wm, ei, ced, n_pad, fl, ew, ch, nch)

    return _tail_call(partial, lin2_w.T, lin2_b.reshape(1, h),
                      lin_w.T, lin_b.reshape(1, h), n, block_n=1000)
